# baseline (speedup 1.0000x reference)
import functools

import jax
import jax.numpy as jnp
from jax import lax
from jax.experimental import pallas as pl
from jax.experimental.pallas import tpu as pltpu
from jax.experimental.pallas import tpu_sc as plsc

_K = 20
_RB = 128


def _scores_body(x_ref, wl_ref, wr_ref, bl_ref, brc_ref,
                 attc_ref, atts_ref, alpha_ref, n, d):
    r = pl.program_id(1)

    xb = x_ref[0]
    xrows = x_ref[0, pl.ds(r * _RB, _RB), :]

    x_l = lax.dot_general(xrows, wl_ref[...],
                          (((1,), (1,)), ((), ())),
                          preferred_element_type=jnp.float32) + bl_ref[...]
    x_rT = lax.dot_general(wr_ref[...], xb,
                           (((1,), (1,)), ((), ())),
                           preferred_element_type=jnp.float32) + brc_ref[...]

    sR = jnp.sum(x_rT * attc_ref[...], axis=0, keepdims=True)

    acc0 = 0.6 * sR + jnp.zeros((_RB, n), jnp.float32)
    acc1 = jnp.zeros((_RB, n), jnp.float32)
    for dd in range(d):
        lcol = x_l[:, dd:dd + 1]
        rrow = x_rT[dd:dd + 1, :]
        a4 = atts_ref[0, dd] * 0.4
        if dd % 2 == 0:
            acc0 = acc0 + a4 * jnp.abs(lcol + rrow)
        else:
            acc1 = acc1 + a4 * jnp.abs(lcol + rrow)
    acc = acc0 + acc1

    finite = (acc * 0.0) == 0.0
    alpha_ref[0] = jnp.where(finite, acc, 0.0)


def _tc_scores(x, W_l, b_l, W_r, b_r, att):
    b, n, in_ch = x.shape
    d = W_l.shape[0]
    nr = n // _RB
    return pl.pallas_call(
        functools.partial(_scores_body, n=n, d=d),
        grid=(b, nr),
        in_specs=[
            pl.BlockSpec((1, n, in_ch), lambda bb, rr: (bb, 0, 0)),
            pl.BlockSpec((d, in_ch), lambda bb, rr: (0, 0)),
            pl.BlockSpec((d, in_ch), lambda bb, rr: (0, 0)),
            pl.BlockSpec((1, d), lambda bb, rr: (0, 0)),
            pl.BlockSpec((d, 1), lambda bb, rr: (0, 0)),
            pl.BlockSpec((d, 1), lambda bb, rr: (0, 0)),
            pl.BlockSpec(memory_space=pltpu.SMEM),
        ],
        out_specs=pl.BlockSpec((1, _RB, n), lambda bb, rr: (bb, rr, 0)),
        out_shape=jax.ShapeDtypeStruct((b, n, n), jnp.float32),
    )(x, W_l, W_r, b_l.reshape(1, d), b_r.reshape(d, 1),
      att.reshape(d, 1), att.reshape(1, d))


def _make_sc_topk(total_rows, n, k, rows_per_batch):
    info = plsc.get_sparse_core_info()
    nw = info.num_cores * info.num_subcores
    rpw = total_rows // nw
    chunks = n // 16
    kp = 32

    mesh = plsc.VectorSubcoreMesh(core_axis_name="c", subcore_axis_name="s")

    @functools.partial(
        pl.kernel, mesh=mesh,
        compiler_params=pltpu.CompilerParams(needs_layout_passes=False),
        out_type=[
            jax.ShapeDtypeStruct((total_rows * kp,), jnp.float32),
            jax.ShapeDtypeStruct((total_rows * kp,), jnp.float32),
        ],
        scratch_types=[
            pltpu.VMEM((rpw * n,), jnp.float32),
            pltpu.VMEM((rpw * kp,), jnp.float32),
            pltpu.VMEM((rpw * kp,), jnp.float32),
        ],
    )
    def sc_topk(alpha_hbm, vals_hbm, idx_hbm, row_v, vbuf, ibuf):
        wid = lax.axis_index("s") * info.num_cores + lax.axis_index("c")
        base_row = wid * rpw
        pltpu.sync_copy(alpha_hbm.at[pl.ds(base_row * n, rpw * n)], row_v)

        lane_i = lax.iota(jnp.int32, 16)
        lane_f = lane_i.astype(jnp.float32)
        vneg = jnp.full((16,), -jnp.inf, jnp.float32)
        lane0 = lane_i == 0
        big = jnp.float32(1e9)
        off_f = ((base_row // rows_per_batch) * rows_per_batch).astype(jnp.float32)

        def extract_row(i):
            rowbase = i * n
            L = row_v[pl.ds(rowbase, 16)]
            for c in range(1, chunks):
                L = jnp.maximum(L, row_v[pl.ds(rowbase + c * 16, 16)])

            valsA = vneg
            valsB = vneg
            idxA = jnp.zeros((16,), jnp.float32)
            idxB = jnp.zeros((16,), jnp.float32)
            m0 = jnp.max(L)

            for t in range(k):
                gm = jnp.max(L)
                lf = jnp.min(jnp.where(L == gm, lane_f, big))
                li = lf.astype(jnp.int32)
                gidx = rowbase + li + 16 * lane_i
                colA = plsc.load_gather(row_v, [gidx])
                colB = plsc.load_gather(row_v, [gidx + 16 * 16])
                candc = jnp.minimum(
                    jnp.where(colA == gm, lane_f, big),
                    jnp.where(colB == gm, lane_f + 16.0, big))
                cf = jnp.min(candc)
                ci = cf.astype(jnp.int32)
                jf = cf * 16.0 + lf
                jloc = ci * 16 + li
                if t < 16:
                    valsA = jnp.where(lane_i == t, gm, valsA)
                    idxA = jnp.where(lane_i == t, jf + off_f, idxA)
                else:
                    valsB = jnp.where(lane_i == (t - 16), gm, valsB)
                    idxB = jnp.where(lane_i == (t - 16), jf + off_f, idxB)
                plsc.store_scatter(row_v,
                                   [jnp.full((16,), rowbase + jloc, jnp.int32)],
                                   vneg, mask=lane0)
                colA2 = jnp.where(lane_i == ci, vneg, colA)
                colB2 = jnp.where(lane_i + 16 == ci, vneg, colB)
                cm = jnp.max(jnp.maximum(colA2, colB2))
                L = jnp.where(lane_i == li, cm, L)

            eA = jnp.where(valsA > -jnp.inf, jnp.exp(valsA - m0), 0.0)
            eB = jnp.where(valsB > -jnp.inf, jnp.exp(valsB - m0), 0.0)
            s = jnp.sum(eA) + jnp.sum(eB)
            sv = jnp.full((16,), s, jnp.float32)
            return eA / sv, eB / sv, idxA, idxB

        @plsc.parallel_loop(0, rpw, unroll=2)
        def _rows(i):
            a0, b0, ia0, ib0 = extract_row(i)
            vbuf[pl.ds(i * kp, 16)] = a0
            vbuf[pl.ds(i * kp + 16, 16)] = b0
            ibuf[pl.ds(i * kp, 16)] = ia0
            ibuf[pl.ds(i * kp + 16, 16)] = ib0
        pltpu.sync_copy(vbuf, vals_hbm.at[pl.ds(base_row * kp, rpw * kp)])
        pltpu.sync_copy(ibuf, idx_hbm.at[pl.ds(base_row * kp, rpw * kp)])

    return sc_topk


@jax.jit
def _feature_graph(x, W_l, b_l, W_r, b_r, att):
    b, n, in_ch = x.shape
    k = min(_K, n)
    alpha = _tc_scores(x, W_l, b_l, W_r, b_r, att)
    sc_topk = _make_sc_topk(b * n, n, k, n)
    vals_f, idx_f = sc_topk(alpha.reshape(-1))
    attn = vals_f.reshape(b * n, 32)[:, :k]
    idxs = idx_f.reshape(b * n, 32)[:, :k]
    return attn, idxs


def kernel(x, edge_index, batch, W_l, b_l, W_r, b_r, att):
    b, n, _ = x.shape
    k = min(_K, n)
    attn, idxs = _feature_graph(x, W_l, b_l, W_r, b_r, att)
    attention = attn.reshape(-1)
    index_j = idxs.astype(jnp.int32).reshape(-1)
    offset = (jnp.arange(b) * n)[:, None]
    index_i = (jnp.tile(jnp.repeat(jnp.arange(n), k), (b,)).reshape(b, -1)
               + offset).reshape(-1)
    new_edge_index = jnp.stack([index_i, index_j], axis=0)
    return new_edge_index, attention


# speedup vs baseline: 1.1245x; 1.1245x over previous
import functools

import jax
import jax.numpy as jnp
from jax import lax
from jax.experimental import pallas as pl
from jax.experimental.pallas import tpu as pltpu
from jax.experimental.pallas import tpu_sc as plsc

_K = 20
_RB = 128


def _scores_body(x_ref, wl_ref, wr_ref, bl_ref, brc_ref,
                 attc_ref, atts_ref, alpha_ref, n, d):
    r = pl.program_id(1)

    xb = x_ref[0]
    xrows = x_ref[0, pl.ds(r * _RB, _RB), :]

    x_l = lax.dot_general(xrows, wl_ref[...],
                          (((1,), (1,)), ((), ())),
                          preferred_element_type=jnp.float32) + bl_ref[...]
    x_rT = lax.dot_general(wr_ref[...], xb,
                           (((1,), (1,)), ((), ())),
                           preferred_element_type=jnp.float32) + brc_ref[...]

    sR = jnp.sum(x_rT * attc_ref[...], axis=0, keepdims=True)

    acc0 = 0.6 * sR + jnp.zeros((_RB, n), jnp.float32)
    acc1 = jnp.zeros((_RB, n), jnp.float32)
    for dd in range(d):
        lcol = x_l[:, dd:dd + 1]
        rrow = x_rT[dd:dd + 1, :]
        a4 = atts_ref[0, dd] * 0.4
        if dd % 2 == 0:
            acc0 = acc0 + a4 * jnp.abs(lcol + rrow)
        else:
            acc1 = acc1 + a4 * jnp.abs(lcol + rrow)
    acc = acc0 + acc1

    finite = (acc * 0.0) == 0.0
    alpha_ref[0] = jnp.where(finite, acc, 0.0)


def _tc_scores(x, W_l, b_l, W_r, b_r, att):
    b, n, in_ch = x.shape
    d = W_l.shape[0]
    nr = n // _RB
    return pl.pallas_call(
        functools.partial(_scores_body, n=n, d=d),
        grid=(b, nr),
        in_specs=[
            pl.BlockSpec((1, n, in_ch), lambda bb, rr: (bb, 0, 0)),
            pl.BlockSpec((d, in_ch), lambda bb, rr: (0, 0)),
            pl.BlockSpec((d, in_ch), lambda bb, rr: (0, 0)),
            pl.BlockSpec((1, d), lambda bb, rr: (0, 0)),
            pl.BlockSpec((d, 1), lambda bb, rr: (0, 0)),
            pl.BlockSpec((d, 1), lambda bb, rr: (0, 0)),
            pl.BlockSpec(memory_space=pltpu.SMEM),
        ],
        out_specs=pl.BlockSpec((1, _RB, n), lambda bb, rr: (bb, rr, 0)),
        out_shape=jax.ShapeDtypeStruct((b, n, n), jnp.float32),
    )(x, W_l, W_r, b_l.reshape(1, d), b_r.reshape(d, 1),
      att.reshape(d, 1), att.reshape(1, d))


def _make_sc_topk(total_rows, n, k, rows_per_batch):
    info = plsc.get_sparse_core_info()
    nw = info.num_cores * info.num_subcores
    rpw = total_rows // nw
    chunks = n // 16
    kp = 32

    mesh = plsc.VectorSubcoreMesh(core_axis_name="c", subcore_axis_name="s")

    @functools.partial(
        pl.kernel, mesh=mesh,
        compiler_params=pltpu.CompilerParams(needs_layout_passes=False),
        out_type=[
            jax.ShapeDtypeStruct((total_rows * kp,), jnp.float32),
            jax.ShapeDtypeStruct((total_rows * kp,), jnp.float32),
        ],
        scratch_types=[
            pltpu.VMEM((rpw * n,), jnp.float32),
            pltpu.VMEM((rpw * kp,), jnp.float32),
            pltpu.VMEM((rpw * kp,), jnp.float32),
        ],
    )
    def sc_topk(alpha_hbm, vals_hbm, idx_hbm, row_v, vbuf, ibuf):
        wid = lax.axis_index("s") * info.num_cores + lax.axis_index("c")
        base_row = wid * rpw
        pltpu.sync_copy(alpha_hbm.at[pl.ds(base_row * n, rpw * n)], row_v)

        lane_i = lax.iota(jnp.int32, 16)
        lane_f = lane_i.astype(jnp.float32)
        vneg = jnp.full((16,), -jnp.inf, jnp.float32)
        lane0 = lane_i == 0
        big = jnp.float32(1e9)
        off_f = ((base_row // rows_per_batch) * rows_per_batch).astype(jnp.float32)

        def extract_row(i):
            rowbase = i * n
            L = row_v[pl.ds(rowbase, 16)]
            for c in range(1, chunks):
                L = jnp.maximum(L, row_v[pl.ds(rowbase + c * 16, 16)])

            valsA = vneg
            valsB = vneg
            idxA = jnp.zeros((16,), jnp.float32)
            idxB = jnp.zeros((16,), jnp.float32)
            m0 = jnp.max(L)

            for t in range(k):
                gm = jnp.max(L)
                lf = jnp.min(jnp.where(L == gm, lane_f, big))
                li = lf.astype(jnp.int32)
                gidx = rowbase + li + 16 * lane_i
                colA = plsc.load_gather(row_v, [gidx])
                colB = plsc.load_gather(row_v, [gidx + 16 * 16])
                candc = jnp.minimum(
                    jnp.where(colA == gm, lane_f, big),
                    jnp.where(colB == gm, lane_f + 16.0, big))
                cf = jnp.min(candc)
                ci = cf.astype(jnp.int32)
                jf = cf * 16.0 + lf
                jloc = ci * 16 + li
                if t < 16:
                    valsA = jnp.where(lane_i == t, gm, valsA)
                    idxA = jnp.where(lane_i == t, jf + off_f, idxA)
                else:
                    valsB = jnp.where(lane_i == (t - 16), gm, valsB)
                    idxB = jnp.where(lane_i == (t - 16), jf + off_f, idxB)
                plsc.store_scatter(row_v,
                                   [jnp.full((16,), rowbase + jloc, jnp.int32)],
                                   vneg, mask=lane0)
                colA2 = jnp.where(lane_i == ci, vneg, colA)
                colB2 = jnp.where(lane_i + 16 == ci, vneg, colB)
                cm = jnp.max(jnp.maximum(colA2, colB2))
                L = jnp.where(lane_i == li, cm, L)

            eA = jnp.where(valsA > -jnp.inf, jnp.exp(valsA - m0), 0.0)
            eB = jnp.where(valsB > -jnp.inf, jnp.exp(valsB - m0), 0.0)
            s = jnp.sum(eA) + jnp.sum(eB)
            sv = jnp.full((16,), s, jnp.float32)
            return eA / sv, eB / sv, idxA, idxB

        @plsc.parallel_loop(0, rpw, unroll=1)
        def _rows(i):
            a0, b0, ia0, ib0 = extract_row(i)
            vbuf[pl.ds(i * kp, 16)] = a0
            vbuf[pl.ds(i * kp + 16, 16)] = b0
            ibuf[pl.ds(i * kp, 16)] = ia0
            ibuf[pl.ds(i * kp + 16, 16)] = ib0
        pltpu.sync_copy(vbuf, vals_hbm.at[pl.ds(base_row * kp, rpw * kp)])
        pltpu.sync_copy(ibuf, idx_hbm.at[pl.ds(base_row * kp, rpw * kp)])

    return sc_topk


@jax.jit
def _feature_graph(x, W_l, b_l, W_r, b_r, att):
    b, n, in_ch = x.shape
    k = min(_K, n)
    alpha = _tc_scores(x, W_l, b_l, W_r, b_r, att)
    sc_topk = _make_sc_topk(b * n, n, k, n)
    vals_f, idx_f = sc_topk(alpha.reshape(-1))
    attn = vals_f.reshape(b * n, 32)[:, :k]
    idxs = idx_f.reshape(b * n, 32)[:, :k]
    return attn, idxs


def kernel(x, edge_index, batch, W_l, b_l, W_r, b_r, att):
    b, n, _ = x.shape
    k = min(_K, n)
    attn, idxs = _feature_graph(x, W_l, b_l, W_r, b_r, att)
    attention = attn.reshape(-1)
    index_j = idxs.astype(jnp.int32).reshape(-1)
    offset = (jnp.arange(b) * n)[:, None]
    index_i = (jnp.tile(jnp.repeat(jnp.arange(n), k), (b,)).reshape(b, -1)
               + offset).reshape(-1)
    new_edge_index = jnp.stack([index_i, index_j], axis=0)
    return new_edge_index, attention


# speedup vs baseline: 1.2112x; 1.0771x over previous
import functools

import jax
import jax.numpy as jnp
from jax import lax
from jax.experimental import pallas as pl
from jax.experimental.pallas import tpu as pltpu
from jax.experimental.pallas import tpu_sc as plsc

_K = 20
_RB = 128


def _scores_body(x_ref, wl_ref, wr_ref, bl_ref, brc_ref,
                 attc_ref, atts_ref, alpha_ref, n, d):
    r = pl.program_id(1)

    xb = x_ref[0]
    xrows = x_ref[0, pl.ds(r * _RB, _RB), :]

    x_l = lax.dot_general(xrows, wl_ref[...],
                          (((1,), (1,)), ((), ())),
                          preferred_element_type=jnp.float32) + bl_ref[...]
    x_rT = lax.dot_general(wr_ref[...], xb,
                           (((1,), (1,)), ((), ())),
                           preferred_element_type=jnp.float32) + brc_ref[...]

    sR = jnp.sum(x_rT * attc_ref[...], axis=0, keepdims=True)

    acc0 = 0.6 * sR + jnp.zeros((_RB, n), jnp.float32)
    acc1 = jnp.zeros((_RB, n), jnp.float32)
    for dd in range(d):
        lcol = x_l[:, dd:dd + 1]
        rrow = x_rT[dd:dd + 1, :]
        a4 = atts_ref[0, dd] * 0.4
        if dd % 2 == 0:
            acc0 = acc0 + a4 * jnp.abs(lcol + rrow)
        else:
            acc1 = acc1 + a4 * jnp.abs(lcol + rrow)
    acc = acc0 + acc1

    finite = (acc * 0.0) == 0.0
    alpha_ref[0] = jnp.where(finite, acc, 0.0)


def _tc_scores(x, W_l, b_l, W_r, b_r, att):
    b, n, in_ch = x.shape
    d = W_l.shape[0]
    nr = n // _RB
    return pl.pallas_call(
        functools.partial(_scores_body, n=n, d=d),
        grid=(b, nr),
        in_specs=[
            pl.BlockSpec((1, n, in_ch), lambda bb, rr: (bb, 0, 0)),
            pl.BlockSpec((d, in_ch), lambda bb, rr: (0, 0)),
            pl.BlockSpec((d, in_ch), lambda bb, rr: (0, 0)),
            pl.BlockSpec((1, d), lambda bb, rr: (0, 0)),
            pl.BlockSpec((d, 1), lambda bb, rr: (0, 0)),
            pl.BlockSpec((d, 1), lambda bb, rr: (0, 0)),
            pl.BlockSpec(memory_space=pltpu.SMEM),
        ],
        out_specs=pl.BlockSpec((1, _RB, n), lambda bb, rr: (bb, rr, 0)),
        out_shape=jax.ShapeDtypeStruct((b, n, n), jnp.float32),
    )(x, W_l, W_r, b_l.reshape(1, d), b_r.reshape(d, 1),
      att.reshape(d, 1), att.reshape(1, d))


def _make_sc_topk(total_rows, n, k, base_offset):
    info = plsc.get_sparse_core_info()
    nw = info.num_cores * info.num_subcores
    rpw = total_rows // nw
    chunks = n // 16
    kp = 32

    mesh = plsc.VectorSubcoreMesh(core_axis_name="c", subcore_axis_name="s")

    @functools.partial(
        pl.kernel, mesh=mesh,
        compiler_params=pltpu.CompilerParams(needs_layout_passes=False),
        out_type=[
            jax.ShapeDtypeStruct((total_rows * kp,), jnp.float32),
            jax.ShapeDtypeStruct((total_rows * kp,), jnp.float32),
        ],
        scratch_types=[
            pltpu.VMEM((rpw * n,), jnp.float32),
            pltpu.VMEM((rpw * kp,), jnp.float32),
            pltpu.VMEM((rpw * kp,), jnp.float32),
        ],
    )
    def sc_topk(alpha_hbm, vals_hbm, idx_hbm, row_v, vbuf, ibuf):
        wid = lax.axis_index("s") * info.num_cores + lax.axis_index("c")
        base_row = wid * rpw
        pltpu.sync_copy(alpha_hbm.at[pl.ds(base_row * n, rpw * n)], row_v)

        lane_i = lax.iota(jnp.int32, 16)
        lane_f = lane_i.astype(jnp.float32)
        vneg = jnp.full((16,), -jnp.inf, jnp.float32)
        lane0 = lane_i == 0
        big = jnp.float32(1e9)
        off_f = jnp.float32(base_offset)

        def extract_row(i):
            rowbase = i * n
            L = row_v[pl.ds(rowbase, 16)]
            for c in range(1, chunks):
                L = jnp.maximum(L, row_v[pl.ds(rowbase + c * 16, 16)])

            valsA = vneg
            valsB = vneg
            idxA = jnp.zeros((16,), jnp.float32)
            idxB = jnp.zeros((16,), jnp.float32)
            m0 = jnp.max(L)

            for t in range(k):
                gm = jnp.max(L)
                lf = jnp.min(jnp.where(L == gm, lane_f, big))
                li = lf.astype(jnp.int32)
                gidx = rowbase + li + 16 * lane_i
                colA = plsc.load_gather(row_v, [gidx])
                colB = plsc.load_gather(row_v, [gidx + 16 * 16])
                candc = jnp.minimum(
                    jnp.where(colA == gm, lane_f, big),
                    jnp.where(colB == gm, lane_f + 16.0, big))
                cf = jnp.min(candc)
                ci = cf.astype(jnp.int32)
                jf = cf * 16.0 + lf
                jloc = ci * 16 + li
                if t < 16:
                    valsA = jnp.where(lane_i == t, gm, valsA)
                    idxA = jnp.where(lane_i == t, jf + off_f, idxA)
                else:
                    valsB = jnp.where(lane_i == (t - 16), gm, valsB)
                    idxB = jnp.where(lane_i == (t - 16), jf + off_f, idxB)
                plsc.store_scatter(row_v,
                                   [jnp.full((16,), rowbase + jloc, jnp.int32)],
                                   vneg, mask=lane0)
                colA2 = jnp.where(lane_i == ci, vneg, colA)
                colB2 = jnp.where(lane_i + 16 == ci, vneg, colB)
                cm = jnp.max(jnp.maximum(colA2, colB2))
                L = jnp.where(lane_i == li, cm, L)

            eA = jnp.where(valsA > -jnp.inf, jnp.exp(valsA - m0), 0.0)
            eB = jnp.where(valsB > -jnp.inf, jnp.exp(valsB - m0), 0.0)
            s = jnp.sum(eA) + jnp.sum(eB)
            sv = jnp.full((16,), s, jnp.float32)
            return eA / sv, eB / sv, idxA, idxB

        @plsc.parallel_loop(0, rpw, unroll=1)
        def _rows(i):
            a0, b0, ia0, ib0 = extract_row(i)
            vbuf[pl.ds(i * kp, 16)] = a0
            vbuf[pl.ds(i * kp + 16, 16)] = b0
            ibuf[pl.ds(i * kp, 16)] = ia0
            ibuf[pl.ds(i * kp + 16, 16)] = ib0
        pltpu.sync_copy(vbuf, vals_hbm.at[pl.ds(base_row * kp, rpw * kp)])
        pltpu.sync_copy(ibuf, idx_hbm.at[pl.ds(base_row * kp, rpw * kp)])

    return sc_topk


@jax.jit
def _feature_graph(x, W_l, b_l, W_r, b_r, att):
    b, n, in_ch = x.shape
    k = min(_K, n)
    vals_parts, idx_parts = [], []
    for bb in range(b):
        xb = lax.slice_in_dim(x, bb, bb + 1, axis=0)
        alpha = _tc_scores(xb, W_l, b_l, W_r, b_r, att)
        sc_topk = _make_sc_topk(n, n, k, bb * n)
        vals_f, idx_f = sc_topk(alpha.reshape(-1))
        vals_parts.append(vals_f)
        idx_parts.append(idx_f)
    vals_all = jnp.concatenate(vals_parts)
    idx_all = jnp.concatenate(idx_parts)
    attn = vals_all.reshape(b * n, 32)[:, :k]
    idxs = idx_all.reshape(b * n, 32)[:, :k]
    return attn, idxs


def kernel(x, edge_index, batch, W_l, b_l, W_r, b_r, att):
    b, n, _ = x.shape
    k = min(_K, n)
    attn, idxs = _feature_graph(x, W_l, b_l, W_r, b_r, att)
    attention = attn.reshape(-1)
    index_j = idxs.astype(jnp.int32).reshape(-1)
    offset = (jnp.arange(b) * n)[:, None]
    index_i = (jnp.tile(jnp.repeat(jnp.arange(n), k), (b,)).reshape(b, -1)
               + offset).reshape(-1)
    new_edge_index = jnp.stack([index_i, index_j], axis=0)
    return new_edge_index, attention


# speedup vs baseline: 1.7027x; 1.4058x over previous
import functools

import jax
import jax.numpy as jnp
from jax import lax
from jax.experimental import pallas as pl
from jax.experimental.pallas import tpu as pltpu
from jax.experimental.pallas import tpu_sc as plsc

_K = 20
_RB = 128


def _scores_body(x_ref, wl_ref, wr_ref, bl_ref, brc_ref,
                 attc_ref, atts_ref, alpha_ref, n, d):
    r = pl.program_id(1)

    xb = x_ref[0]
    xrows = x_ref[0, pl.ds(r * _RB, _RB), :]

    x_l = lax.dot_general(xrows, wl_ref[...],
                          (((1,), (1,)), ((), ())),
                          preferred_element_type=jnp.float32) + bl_ref[...]
    x_rT = lax.dot_general(wr_ref[...], xb,
                           (((1,), (1,)), ((), ())),
                           preferred_element_type=jnp.float32) + brc_ref[...]

    sR = jnp.sum(x_rT * attc_ref[...], axis=0, keepdims=True)

    acc0 = 0.6 * sR + jnp.zeros((_RB, n), jnp.float32)
    acc1 = jnp.zeros((_RB, n), jnp.float32)
    for dd in range(d):
        lcol = x_l[:, dd:dd + 1]
        rrow = x_rT[dd:dd + 1, :]
        a4 = atts_ref[0, dd] * 0.4
        if dd % 2 == 0:
            acc0 = acc0 + a4 * jnp.abs(lcol + rrow)
        else:
            acc1 = acc1 + a4 * jnp.abs(lcol + rrow)
    acc = acc0 + acc1

    finite = (acc * 0.0) == 0.0
    alpha_ref[0] = jnp.where(finite, acc, 0.0)


def _tc_scores(x, W_l, b_l, W_r, b_r, att):
    b, n, in_ch = x.shape
    d = W_l.shape[0]
    nr = n // _RB
    return pl.pallas_call(
        functools.partial(_scores_body, n=n, d=d),
        grid=(b, nr),
        in_specs=[
            pl.BlockSpec((1, n, in_ch), lambda bb, rr: (bb, 0, 0)),
            pl.BlockSpec((d, in_ch), lambda bb, rr: (0, 0)),
            pl.BlockSpec((d, in_ch), lambda bb, rr: (0, 0)),
            pl.BlockSpec((1, d), lambda bb, rr: (0, 0)),
            pl.BlockSpec((d, 1), lambda bb, rr: (0, 0)),
            pl.BlockSpec((d, 1), lambda bb, rr: (0, 0)),
            pl.BlockSpec(memory_space=pltpu.SMEM),
        ],
        out_specs=pl.BlockSpec((1, _RB, n), lambda bb, rr: (bb, rr, 0)),
        out_shape=jax.ShapeDtypeStruct((b, n, n), jnp.float32),
    )(x, W_l, W_r, b_l.reshape(1, d), b_r.reshape(d, 1),
      att.reshape(d, 1), att.reshape(1, d))


def _make_sc_topk(total_rows, n, k, base_offset):
    info = plsc.get_sparse_core_info()
    nw = info.num_cores * info.num_subcores
    rpw = total_rows // nw
    chunks = n // 16
    kp = 32

    mesh = plsc.VectorSubcoreMesh(core_axis_name="c", subcore_axis_name="s")

    @functools.partial(
        pl.kernel, mesh=mesh,
        compiler_params=pltpu.CompilerParams(needs_layout_passes=False),
        out_type=[
            jax.ShapeDtypeStruct((total_rows * kp,), jnp.float32),
            jax.ShapeDtypeStruct((total_rows * kp,), jnp.int32),
        ],
        scratch_types=[
            pltpu.VMEM((rpw * n,), jnp.float32),
            pltpu.VMEM((rpw * kp,), jnp.float32),
            pltpu.VMEM((rpw * kp,), jnp.int32),
        ],
    )
    def sc_topk(alpha_hbm, vals_hbm, idx_hbm, row_v, vbuf, ibuf):
        wid = lax.axis_index("s") * info.num_cores + lax.axis_index("c")
        base_row = wid * rpw
        pltpu.sync_copy(alpha_hbm.at[pl.ds(base_row * n, rpw * n)], row_v)

        lane_i = lax.iota(jnp.int32, 16)
        vneg = jnp.full((16,), -jnp.inf, jnp.float32)
        lane0 = lane_i == 0
        off_i = jnp.int32(base_offset)

        def extract_row(i):
            rowbase = i * n
            L = row_v[pl.ds(rowbase, 16)]
            for c in range(1, chunks):
                L = jnp.maximum(L, row_v[pl.ds(rowbase + c * 16, 16)])

            valsA = vneg
            valsB = vneg
            idxA = jnp.zeros((16,), jnp.int32)
            idxB = jnp.zeros((16,), jnp.int32)
            m0 = jnp.float32(0.0)

            for t in range(k):
                gm = jnp.max(L)
                if t == 0:
                    m0 = gm
                li = plsc.all_reduce_ffs(L == gm)
                gidx = rowbase + li + 16 * lane_i
                colA = plsc.load_gather(row_v, [gidx])
                colB = plsc.load_gather(row_v, [gidx + 256])
                ffsA = plsc.all_reduce_ffs(colA == gm)
                ffsB = plsc.all_reduce_ffs(colB == gm)
                ci = jnp.where(ffsA < 16, ffsA, 16 + ffsB)
                jidx = ci * 16 + li + off_i
                if t < 16:
                    valsA = jnp.where(lane_i == t, gm, valsA)
                    idxA = jnp.where(lane_i == t, jidx, idxA)
                else:
                    valsB = jnp.where(lane_i == (t - 16), gm, valsB)
                    idxB = jnp.where(lane_i == (t - 16), jidx, idxB)
                plsc.store_scatter(row_v, [rowbase + ci * 16 + li],
                                   vneg, mask=lane0)
                colA2 = jnp.where(lane_i == ci, vneg, colA)
                colB2 = jnp.where(lane_i + 16 == ci, vneg, colB)
                cm = jnp.max(jnp.maximum(colA2, colB2))
                L = jnp.where(lane_i == li, cm, L)

            eA = jnp.where(valsA > -jnp.inf, jnp.exp(valsA - m0), 0.0)
            eB = jnp.where(valsB > -jnp.inf, jnp.exp(valsB - m0), 0.0)
            s = jnp.sum(eA) + jnp.sum(eB)
            sv = jnp.full((16,), s, jnp.float32)
            return eA / sv, eB / sv, idxA, idxB

        @plsc.parallel_loop(0, rpw, unroll=1)
        def _rows(i):
            a0, b0, ia0, ib0 = extract_row(i)
            vbuf[pl.ds(i * kp, 16)] = a0
            vbuf[pl.ds(i * kp + 16, 16)] = b0
            ibuf[pl.ds(i * kp, 16)] = ia0
            ibuf[pl.ds(i * kp + 16, 16)] = ib0
        pltpu.sync_copy(vbuf, vals_hbm.at[pl.ds(base_row * kp, rpw * kp)])
        pltpu.sync_copy(ibuf, idx_hbm.at[pl.ds(base_row * kp, rpw * kp)])

    return sc_topk


@jax.jit
def _feature_graph(x, W_l, b_l, W_r, b_r, att):
    b, n, in_ch = x.shape
    k = min(_K, n)
    vals_parts, idx_parts = [], []
    for bb in range(b):
        xb = lax.slice_in_dim(x, bb, bb + 1, axis=0)
        alpha = _tc_scores(xb, W_l, b_l, W_r, b_r, att)
        sc_topk = _make_sc_topk(n, n, k, bb * n)
        vals_f, idx_f = sc_topk(alpha.reshape(-1))
        vals_parts.append(vals_f)
        idx_parts.append(idx_f)
    vals_all = jnp.concatenate(vals_parts)
    idx_all = jnp.concatenate(idx_parts)
    attn = vals_all.reshape(b * n, 32)[:, :k]
    idxs = idx_all.reshape(b * n, 32)[:, :k]
    return attn, idxs


def kernel(x, edge_index, batch, W_l, b_l, W_r, b_r, att):
    b, n, _ = x.shape
    k = min(_K, n)
    attn, idxs = _feature_graph(x, W_l, b_l, W_r, b_r, att)
    attention = attn.reshape(-1)
    index_j = idxs.astype(jnp.int32).reshape(-1)
    offset = (jnp.arange(b) * n)[:, None]
    index_i = (jnp.tile(jnp.repeat(jnp.arange(n), k), (b,)).reshape(b, -1)
               + offset).reshape(-1)
    new_edge_index = jnp.stack([index_i, index_j], axis=0)
    return new_edge_index, attention
